# baseline (speedup 1.0000x reference)
import jax
import jax.numpy as jnp
from jax.experimental import pallas as pl
from jax.experimental.pallas import tpu as pltpu

_BM = 256
_NBUF = 4


def _body(x_ref, a_hbm, o_ref, abuf, sems):
    i = pl.program_id(0)
    steps = pl.num_programs(0)
    slot = jax.lax.rem(i, _NBUF)

    @pl.when(i == 0)
    def _prologue():
        for j in range(_NBUF):
            pltpu.make_async_copy(
                a_hbm.at[pl.ds(j * _BM, _BM), :], abuf.at[j], sems.at[j]
            ).start()

    pltpu.make_async_copy(
        a_hbm.at[pl.ds(i * _BM, _BM), :], abuf.at[slot], sems.at[slot]
    ).wait()

    t = jnp.dot(abuf[slot], x_ref[...], preferred_element_type=jnp.float32)
    o_ref[...] = (t > 0.5).astype(jnp.float32)

    @pl.when(i + _NBUF < steps)
    def _prefetch():
        nxt = i + _NBUF
        pltpu.make_async_copy(
            a_hbm.at[pl.ds(nxt * _BM, _BM), :], abuf.at[slot], sems.at[slot]
        ).start()


def kernel(x, a):
    m, k = a.shape
    n = x.shape[1]
    return pl.pallas_call(
        _body,
        grid=(m // _BM,),
        in_specs=[
            pl.BlockSpec((k, n), lambda i: (0, 0)),
            pl.BlockSpec(memory_space=pltpu.MemorySpace.HBM),
        ],
        out_specs=pl.BlockSpec((_BM, n), lambda i: (i, 0)),
        out_shape=jax.ShapeDtypeStruct((m, n), jnp.float32),
        scratch_shapes=[
            pltpu.VMEM((_NBUF, _BM, 8192), jnp.float32),
            pltpu.SemaphoreType.DMA((_NBUF,)),
        ],
        compiler_params=pltpu.CompilerParams(
            dimension_semantics=("arbitrary",),
        ),
    )(x, a)
